# baseline (speedup 1.0000x reference)
import jax
import jax.numpy as jnp
from jax import lax
from jax.experimental import pallas as pl
from jax.experimental.pallas import tpu as pltpu
from jax.experimental.pallas import tpu_sc as plsc

B = 32768
D = 2048
NC = 2
NS = 16
NW = NC * NS
BPW = B // NW
C = 8
NCHUNK = BPW // C


def _emb_body(idx_hbm, table_hbm, out_hbm, idx_v, buf0, buf1,
              gsem0, gsem1, osem0, osem1):
    wid = lax.axis_index("s") * NC + lax.axis_index("c")
    base = wid * BPW
    pltpu.sync_copy(idx_hbm.at[pl.ds(base, BPW)], idx_v)

    bufs = (buf0, buf1)
    gsems = (gsem0, gsem1)
    osems = (osem0, osem1)

    def gather(g, b):
        return pltpu.make_async_copy(
            table_hbm.at[idx_v.at[pl.ds(g * C, C)]], bufs[b], gsems[b])

    def store(g, b):
        return pltpu.make_async_copy(
            bufs[b], out_hbm.at[pl.ds(base + g * C, C)], osems[b])

    gather(0, 0).start()
    gather(1, 1).start()

    def step(h, last):
        for b in range(2):
            g = 2 * h + b
            gather(g, b).wait()
            store(g, b).start()
            store(g, b).wait()
            if not last:
                gather(g + 2, b).start()

    def body(h, carry):
        step(h, last=False)
        return carry

    lax.fori_loop(0, NCHUNK // 2 - 1, body, 0)
    step(NCHUNK // 2 - 1, last=True)


_emb = pl.kernel(
    _emb_body,
    out_type=jax.ShapeDtypeStruct((B, D), jnp.float32),
    mesh=plsc.VectorSubcoreMesh(core_axis_name="c", subcore_axis_name="s"),
    scratch_types=[
        pltpu.VMEM((BPW,), jnp.int32),
        pltpu.VMEM((C, D), jnp.float32),
        pltpu.VMEM((C, D), jnp.float32),
        pltpu.SemaphoreType.DMA,
        pltpu.SemaphoreType.DMA,
        pltpu.SemaphoreType.DMA,
        pltpu.SemaphoreType.DMA,
    ],
)


def kernel(position_ids, weight):
    idx = position_ids.reshape(-1).astype(jnp.int32)
    out = _emb(idx, weight)
    return out.reshape(position_ids.shape + (weight.shape[1],))


# speedup vs baseline: 1.0141x; 1.0141x over previous
import jax
import jax.numpy as jnp
from jax import lax
from jax.experimental import pallas as pl
from jax.experimental.pallas import tpu as pltpu
from jax.experimental.pallas import tpu_sc as plsc

B = 32768
D = 2048
NC = 2
NS = 16
NW = NC * NS
BPW = B // NW
C = 16
NCHUNK = BPW // C


def _emb_body(idx_hbm, table_hbm, out_hbm, idx_v, buf0, buf1,
              gsem0, gsem1, osem0, osem1):
    wid = lax.axis_index("c") * NS + lax.axis_index("s")
    base = wid * BPW
    pltpu.sync_copy(idx_hbm.at[pl.ds(base, BPW)], idx_v)

    bufs = (buf0, buf1)
    gsems = (gsem0, gsem1)
    osems = (osem0, osem1)

    def gather(g, b):
        return pltpu.make_async_copy(
            table_hbm.at[idx_v.at[pl.ds(g * C, C)]], bufs[b], gsems[b])

    def store(g, b):
        return pltpu.make_async_copy(
            bufs[b], out_hbm.at[pl.ds(base + g * C, C)], osems[b])

    gather(0, 0).start()
    gather(1, 1).start()

    def step(h, last):
        for b in range(2):
            g = 2 * h + b
            gather(g, b).wait()
            store(g, b).start()
            store(g, b).wait()
            if not last:
                gather(g + 2, b).start()

    def body(h, carry):
        step(h, last=False)
        return carry

    lax.fori_loop(0, NCHUNK // 2 - 1, body, 0)
    step(NCHUNK // 2 - 1, last=True)


_emb = pl.kernel(
    _emb_body,
    out_type=jax.ShapeDtypeStruct((B, D), jnp.float32),
    mesh=plsc.VectorSubcoreMesh(core_axis_name="c", subcore_axis_name="s"),
    scratch_types=[
        pltpu.VMEM((BPW,), jnp.int32),
        pltpu.VMEM((C, D), jnp.float32),
        pltpu.VMEM((C, D), jnp.float32),
        pltpu.SemaphoreType.DMA,
        pltpu.SemaphoreType.DMA,
        pltpu.SemaphoreType.DMA,
        pltpu.SemaphoreType.DMA,
    ],
)


def kernel(position_ids, weight):
    idx = position_ids.reshape(-1).astype(jnp.int32)
    out = _emb(idx, weight)
    return out.reshape(position_ids.shape + (weight.shape[1],))


# speedup vs baseline: 1.0151x; 1.0010x over previous
import jax
import jax.numpy as jnp
from jax import lax
from jax.experimental import pallas as pl
from jax.experimental.pallas import tpu as pltpu
from jax.experimental.pallas import tpu_sc as plsc

B = 32768
D = 2048
NC = 2
NS = 16
NW = NC * NS
BPW = B // NW
C = 24
NMAIN = 42
CT = BPW - NMAIN * C


def _emb_body(idx_hbm, table_hbm, out_hbm, idx_v, buf0, buf1,
              gsem0, gsem1, osem0, osem1):
    wid = lax.axis_index("c") * NS + lax.axis_index("s")
    base = wid * BPW
    pltpu.sync_copy(idx_hbm.at[pl.ds(base, BPW)], idx_v)

    bufs = (buf0, buf1)
    gsems = (gsem0, gsem1)
    osems = (osem0, osem1)

    def gather(off, n, b):
        return pltpu.make_async_copy(
            table_hbm.at[idx_v.at[pl.ds(off, n)]],
            bufs[b].at[pl.ds(0, n)], gsems[b])

    def store(off, n, b):
        return pltpu.make_async_copy(
            bufs[b].at[pl.ds(0, n)], out_hbm.at[pl.ds(base + off, n)],
            osems[b])

    gather(0, C, 0).start()
    gather(C, C, 1).start()

    def step(g, b, n_next):
        gather(g * C, C, b).wait()
        store(g * C, C, b).start()
        store(g * C, C, b).wait()
        if n_next:
            gather((g + 2) * C, n_next, b).start()

    def body(h, carry):
        step(2 * h, 0, C)
        step(2 * h + 1, 1, C)
        return carry

    lax.fori_loop(0, NMAIN // 2 - 1, body, 0)
    gather(40 * C, C, 0).wait()
    store(40 * C, C, 0).start()
    store(40 * C, C, 0).wait()
    gather(NMAIN * C, CT, 0).start()
    gather(41 * C, C, 1).wait()
    store(41 * C, C, 1).start()
    store(41 * C, C, 1).wait()
    gather(NMAIN * C, CT, 0).wait()
    store(NMAIN * C, CT, 0).start()
    store(NMAIN * C, CT, 0).wait()


_emb = pl.kernel(
    _emb_body,
    out_type=jax.ShapeDtypeStruct((B, D), jnp.float32),
    mesh=plsc.VectorSubcoreMesh(core_axis_name="c", subcore_axis_name="s"),
    scratch_types=[
        pltpu.VMEM((BPW,), jnp.int32),
        pltpu.VMEM((C, D), jnp.float32),
        pltpu.VMEM((C, D), jnp.float32),
        pltpu.SemaphoreType.DMA,
        pltpu.SemaphoreType.DMA,
        pltpu.SemaphoreType.DMA,
        pltpu.SemaphoreType.DMA,
    ],
)


def kernel(position_ids, weight):
    idx = position_ids.reshape(-1).astype(jnp.int32)
    out = _emb(idx, weight)
    return out.reshape(position_ids.shape + (weight.shape[1],))
